# baseline (speedup 1.0000x reference)
import jax
import jax.numpy as jnp
from jax import lax
from jax.experimental import pallas as pl
from jax.experimental.pallas import tpu as pltpu
from jax.experimental.pallas import tpu_sc as plsc

_N = 10000
_E = 320000
_D = 128
_R = 8
_NB = 4

_NC = 2
_NS = 16
_NW = _NC * _NS
_CH = 80
_CHUNKS = 128
_C0 = 128
_C1 = 2 * _CHUNKS - _C0
_PER_TILE = _CHUNKS * _CH
_E_PAD = _PER_TILE * _NW
_NT = _NW * _CHUNKS
_N_PAD = 10240
_RPT = _N_PAD // _NS
_NBUF = 4

_BN = 400
_G1 = _N // _BN


def _tc_pre_body(comp_ref, h_ref, basis_ref, loopw_ref, hw_ref, loopp_ref, w_s):
    @pl.when(pl.program_id(0) == 0)
    def _():
        for r in range(_R):
            w = comp_ref[r, 0] * basis_ref[0]
            for b in range(1, _NB):
                w = w + comp_ref[r, b] * basis_ref[b]
            w_s[r] = w

    h = h_ref[...]
    for r in range(_R):
        v = jnp.dot(h, w_s[r], preferred_element_type=jnp.float32)
        hw_ref[0, r] = v
        hw_ref[1, r] = v
    loopp_ref[...] = jnp.dot(h, loopw_ref[...], preferred_element_type=jnp.float32)


def _tc_pre(comp, h, basis, loop_weight):
    return pl.pallas_call(
        _tc_pre_body,
        grid=(_G1,),
        in_specs=[
            pl.BlockSpec(memory_space=pltpu.SMEM),
            pl.BlockSpec((_BN, _D), lambda i: (i, 0)),
            pl.BlockSpec((_NB, _D, _D), lambda i: (0, 0, 0)),
            pl.BlockSpec((_D, _D), lambda i: (0, 0)),
        ],
        out_specs=[
            pl.BlockSpec((_NC, _R, _BN, _D), lambda i: (0, 0, i, 0)),
            pl.BlockSpec((_BN, _D), lambda i: (i, 0)),
        ],
        out_shape=[
            jax.ShapeDtypeStruct((_NC, _R, _N, _D), jnp.float32),
            jax.ShapeDtypeStruct((_N, _D), jnp.float32),
        ],
        scratch_shapes=[pltpu.VMEM((_R, _D, _D), jnp.float32)],
    )(comp, h, basis, loop_weight)


def _sc_edge_body(edata_hbm, nrm_hbm, hw_hbm, out_hbm,
                  ebuf_v, nrm_v, gidx_v, rows_v, acc_sh,
                  sg0, sg1, sg2, sg3, ss0, ss1, ss2, ss3):
    c = lax.axis_index("c")
    s = lax.axis_index("s")
    cbase = jnp.where(c == 0, s * _C0, _NS * _C0 + s * _C1)
    nit = jnp.where(c == 0, _C0 // _NBUF, _C1 // _NBUF)
    sem_g = (sg0, sg1, sg2, sg3)
    sem_s = (ss0, ss1, ss2, ss3)

    def _load_and_gather(g, j):
        pltpu.sync_copy(edata_hbm.at[cbase + g], ebuf_v.at[j])
        pltpu.sync_copy(nrm_hbm.at[cbase + g], nrm_v.at[j])

        def _gix(i, cc):
            sl = pl.ds(i * 16, 16)
            gidx_v[j, sl] = ebuf_v[j, 1, sl] * _N + ebuf_v[j, 0, sl]
            return cc

        lax.fori_loop(0, _CH // 16, _gix, 0)
        pltpu.async_copy(hw_hbm.at[c].at[gidx_v.at[j]], rows_v.at[j], sem_g[j])

    _load_and_gather(0, 0)
    _load_and_gather(1, 1)

    zero16 = jnp.zeros((16,), jnp.float32)

    def _zrow(e, carry):
        for q in range(_D // 16):
            rows_v[3, e, pl.ds(q * 16, 16)] = zero16
        return carry

    lax.fori_loop(0, _CH, _zrow, 0)
    r0 = s * _RPT
    for k in range(_RPT // _CH):
        pltpu.sync_copy(rows_v.at[3], acc_sh.at[pl.ds(r0 + k * _CH, _CH)])
    plsc.subcore_barrier()

    def _block(it, b):
        g = it * _NBUF + b
        pltpu.make_async_copy(hw_hbm.at[c].at[gidx_v.at[b]], rows_v.at[b],
                              sem_g[b]).wait()

        def _scale(gg, cc):
            nv = nrm_v[b, pl.ds(gg * 16, 16)]
            for jj in range(16):
                e = gg * 16 + jj
                nsp = nv.at[jnp.full((16,), jj, jnp.int32)].get(
                    mode='promise_in_bounds')
                for q in range(_D // 16):
                    sl = pl.ds(q * 16, 16)
                    rows_v[b, e, sl] = rows_v[b, e, sl] * nsp
            return cc

        lax.fori_loop(0, _CH // 16, _scale, 0)
        pltpu.async_copy(rows_v.at[b], acc_sh.at[ebuf_v.at[b, 2]], sem_s[b],
                         add=True)

        jp = (b + 2) % _NBUF

        def _prefetch():
            _load_and_gather(g + 2, jp)

        def _drain_then_prefetch():
            pltpu.make_async_copy(rows_v.at[jp], acc_sh.at[ebuf_v.at[jp, 2]],
                                  sem_s[jp]).wait()
            _prefetch()

        if b >= 2:
            @pl.when(it < nit - 1)
            def _():
                _drain_then_prefetch()
        else:
            @pl.when(it == 0)
            def _():
                _prefetch()

            @pl.when(it > 0)
            def _():
                _drain_then_prefetch()

    def _ring(it, cc):
        for b in range(_NBUF):
            _block(it, b)
        return cc

    lax.fori_loop(0, nit, _ring, 0)

    for j in range(_NBUF):
        pltpu.make_async_copy(rows_v.at[j], acc_sh.at[ebuf_v.at[j, 2]],
                              sem_s[j]).wait()

    plsc.subcore_barrier()
    pltpu.sync_copy(acc_sh.at[pl.ds(r0, _RPT)], out_hbm.at[c, pl.ds(r0, _RPT)])


def _sc_edge(edata, nrm2d, hw_flat):
    k = pl.kernel(
        _sc_edge_body,
        out_type=jax.ShapeDtypeStruct((_NC, _N_PAD, _D), jnp.float32),
        mesh=plsc.VectorSubcoreMesh(core_axis_name="c", subcore_axis_name="s"),
        scratch_types=[
            pltpu.VMEM((_NBUF, 3, _CH), jnp.int32),
            pltpu.VMEM((_NBUF, _CH), jnp.float32),
            pltpu.VMEM((_NBUF, _CH), jnp.int32),
            pltpu.VMEM((_NBUF, _CH, _D), jnp.float32),
            pltpu.VMEM_SHARED((_N_PAD, _D), jnp.float32),
        ] + [pltpu.SemaphoreType.DMA] * (2 * _NBUF),
    )
    return k(edata, nrm2d, hw_flat)


def _tc_post_body(p_ref, loopp_ref, h_ref, ffnw_ref, hb_ref, fb_ref,
                  lg_ref, lb_ref, out_ref):
    node = p_ref[0] + p_ref[1] + loopp_ref[...] + hb_ref[...]
    node = jnp.maximum(node, 0.0)
    y = lax.dot_general(node, ffnw_ref[...], (((1,), (1,)), ((), ())),
                        preferred_element_type=jnp.float32)
    y = y + fb_ref[...] + h_ref[...]
    mu = jnp.mean(y, axis=1, keepdims=True)
    d = y - mu
    var = jnp.mean(d * d, axis=1, keepdims=True)
    out_ref[...] = d * lax.rsqrt(var + 1e-8) * lg_ref[...] + lb_ref[...]


def _tc_post(partials, loopp, h, ffn_W, h_bias, ffn_b, ln_gamma, ln_beta):
    vec = pl.BlockSpec((1, _D), lambda i: (0, 0))
    return pl.pallas_call(
        _tc_post_body,
        grid=(_G1,),
        in_specs=[
            pl.BlockSpec((_NC, _BN, _D), lambda i: (0, i, 0)),
            pl.BlockSpec((_BN, _D), lambda i: (i, 0)),
            pl.BlockSpec((_BN, _D), lambda i: (i, 0)),
            pl.BlockSpec((_D, _D), lambda i: (0, 0)),
            vec, vec, vec, vec,
        ],
        out_specs=pl.BlockSpec((_BN, _D), lambda i: (i, 0)),
        out_shape=jax.ShapeDtypeStruct((_N, _D), jnp.float32),
    )(partials, loopp, h, ffn_W, h_bias, ffn_b, ln_gamma, ln_beta)


def kernel(initial_embeddings, edge_index, etype, norm, basis, comp,
           loop_weight, h_bias, ffn_W, ffn_b, ln_gamma, ln_beta):
    h = initial_embeddings
    pad = _E_PAD - _E
    src_p = jnp.pad(edge_index[0], (0, pad))
    et_p = jnp.pad(etype, (0, pad))
    dst_p = jnp.pad(edge_index[1], (0, pad))
    nrm_p = jnp.pad(norm[:, 0], (0, pad))
    edata = jnp.stack([src_p, et_p, dst_p]).reshape(3, _NT, _CH).transpose(1, 0, 2)
    nrm2d = nrm_p.reshape(_NT, _CH)

    hw, loopp = _tc_pre(comp, h, basis, loop_weight)
    partials = _sc_edge(edata, nrm2d, hw.reshape(_NC, _R * _N, _D))
    return _tc_post(partials, loopp, h, ffn_W,
                    h_bias.reshape(1, _D), ffn_b.reshape(1, _D),
                    ln_gamma.reshape(1, _D), ln_beta.reshape(1, _D))


# speedup vs baseline: 1.1910x; 1.1910x over previous
import jax
import jax.numpy as jnp
from jax import lax
from jax.experimental import pallas as pl
from jax.experimental.pallas import tpu as pltpu
from jax.experimental.pallas import tpu_sc as plsc

_N = 10000
_E = 320000
_D = 128
_R = 8
_NB = 4

_NC = 2
_NS = 16
_NW = _NC * _NS
_CH = 80
_CHUNKS = 128
_C0 = 240
_C1 = 2 * _CHUNKS - _C0
_PER_TILE = _CHUNKS * _CH
_E_PAD = _PER_TILE * _NW
_NT = _NW * _CHUNKS
_N_PAD = 10240
_RPT = _N_PAD // _NS
_NBUF = 4

_BN = 400
_G1 = _N // _BN


def _tc_pre_body(comp_ref, h_ref, basis_ref, loopw_ref, hw_ref, loopp_ref, w_s):
    @pl.when(pl.program_id(0) == 0)
    def _():
        for r in range(_R):
            w = comp_ref[r, 0] * basis_ref[0]
            for b in range(1, _NB):
                w = w + comp_ref[r, b] * basis_ref[b]
            w_s[r] = w

    h = h_ref[...]
    for r in range(_R):
        hw_ref[r] = jnp.dot(h, w_s[r], preferred_element_type=jnp.float32)
    loopp_ref[...] = jnp.dot(h, loopw_ref[...], preferred_element_type=jnp.float32)


def _tc_pre(comp, h, basis, loop_weight):
    return pl.pallas_call(
        _tc_pre_body,
        grid=(_G1,),
        in_specs=[
            pl.BlockSpec(memory_space=pltpu.SMEM),
            pl.BlockSpec((_BN, _D), lambda i: (i, 0)),
            pl.BlockSpec((_NB, _D, _D), lambda i: (0, 0, 0)),
            pl.BlockSpec((_D, _D), lambda i: (0, 0)),
        ],
        out_specs=[
            pl.BlockSpec((_R, _BN, _D), lambda i: (0, i, 0)),
            pl.BlockSpec((_BN, _D), lambda i: (i, 0)),
        ],
        out_shape=[
            jax.ShapeDtypeStruct((_R, _N, _D), jnp.float32),
            jax.ShapeDtypeStruct((_N, _D), jnp.float32),
        ],
        scratch_shapes=[pltpu.VMEM((_R, _D, _D), jnp.float32)],
    )(comp, h, basis, loop_weight)


def _sc_edge_body(edata_hbm, nrm_hbm, hw_hbm, out_hbm,
                  ebuf_v, nrm_v, gidx_v, rows_v, acc_sh,
                  sg0, sg1, sg2, sg3, ss0, ss1, ss2, ss3):
    c = lax.axis_index("c")
    s = lax.axis_index("s")
    cbase = jnp.where(c == 0, s * _C0, _NS * _C0 + s * _C1)
    nit = jnp.where(c == 0, _C0 // _NBUF, _C1 // _NBUF)
    sem_g = (sg0, sg1, sg2, sg3)
    sem_s = (ss0, ss1, ss2, ss3)

    def _load_and_gather(g, j):
        pltpu.sync_copy(edata_hbm.at[cbase + g], ebuf_v.at[j])
        pltpu.sync_copy(nrm_hbm.at[cbase + g], nrm_v.at[j])

        def _gix(i, cc):
            sl = pl.ds(i * 16, 16)
            gidx_v[j, sl] = ebuf_v[j, 1, sl] * _N + ebuf_v[j, 0, sl]
            return cc

        lax.fori_loop(0, _CH // 16, _gix, 0)
        pltpu.async_copy(hw_hbm.at[gidx_v.at[j]], rows_v.at[j], sem_g[j])

    _load_and_gather(0, 0)
    _load_and_gather(1, 1)

    zero16 = jnp.zeros((16,), jnp.float32)

    def _zrow(e, carry):
        for q in range(_D // 16):
            rows_v[3, e, pl.ds(q * 16, 16)] = zero16
        return carry

    lax.fori_loop(0, _CH, _zrow, 0)
    r0 = s * _RPT
    for k in range(_RPT // _CH):
        pltpu.sync_copy(rows_v.at[3], acc_sh.at[pl.ds(r0 + k * _CH, _CH)])
    plsc.subcore_barrier()

    def _block(it, b):
        g = it * _NBUF + b
        pltpu.make_async_copy(hw_hbm.at[gidx_v.at[b]], rows_v.at[b],
                              sem_g[b]).wait()

        def _scale(gg, cc):
            nv = nrm_v[b, pl.ds(gg * 16, 16)]
            for jj in range(16):
                e = gg * 16 + jj
                nsp = nv.at[jnp.full((16,), jj, jnp.int32)].get(
                    mode='promise_in_bounds')
                for q in range(_D // 16):
                    sl = pl.ds(q * 16, 16)
                    rows_v[b, e, sl] = rows_v[b, e, sl] * nsp
            return cc

        lax.fori_loop(0, _CH // 16, _scale, 0)
        pltpu.async_copy(rows_v.at[b], acc_sh.at[ebuf_v.at[b, 2]], sem_s[b],
                         add=True)

        jp = (b + 2) % _NBUF

        def _prefetch():
            _load_and_gather(g + 2, jp)

        def _drain_then_prefetch():
            pltpu.make_async_copy(rows_v.at[jp], acc_sh.at[ebuf_v.at[jp, 2]],
                                  sem_s[jp]).wait()
            _prefetch()

        if b >= 2:
            @pl.when(it < nit - 1)
            def _():
                _drain_then_prefetch()
        else:
            @pl.when(it == 0)
            def _():
                _prefetch()

            @pl.when(it > 0)
            def _():
                _drain_then_prefetch()

    def _ring(it, cc):
        for b in range(_NBUF):
            _block(it, b)
        return cc

    lax.fori_loop(0, nit, _ring, 0)

    for j in range(_NBUF):
        pltpu.make_async_copy(rows_v.at[j], acc_sh.at[ebuf_v.at[j, 2]],
                              sem_s[j]).wait()

    plsc.subcore_barrier()
    pltpu.sync_copy(acc_sh.at[pl.ds(r0, _RPT)], out_hbm.at[c, pl.ds(r0, _RPT)])


def _sc_edge(edata, nrm2d, hw_flat):
    k = pl.kernel(
        _sc_edge_body,
        out_type=jax.ShapeDtypeStruct((_NC, _N_PAD, _D), jnp.float32),
        mesh=plsc.VectorSubcoreMesh(core_axis_name="c", subcore_axis_name="s"),
        scratch_types=[
            pltpu.VMEM((_NBUF, 3, _CH), jnp.int32),
            pltpu.VMEM((_NBUF, _CH), jnp.float32),
            pltpu.VMEM((_NBUF, _CH), jnp.int32),
            pltpu.VMEM((_NBUF, _CH, _D), jnp.float32),
            pltpu.VMEM_SHARED((_N_PAD, _D), jnp.float32),
        ] + [pltpu.SemaphoreType.DMA] * (2 * _NBUF),
    )
    return k(edata, nrm2d, hw_flat)


def _tc_post_body(p_ref, loopp_ref, h_ref, ffnw_ref, hb_ref, fb_ref,
                  lg_ref, lb_ref, out_ref):
    node = p_ref[0] + p_ref[1] + loopp_ref[...] + hb_ref[...]
    node = jnp.maximum(node, 0.0)
    y = lax.dot_general(node, ffnw_ref[...], (((1,), (1,)), ((), ())),
                        preferred_element_type=jnp.float32)
    y = y + fb_ref[...] + h_ref[...]
    mu = jnp.mean(y, axis=1, keepdims=True)
    d = y - mu
    var = jnp.mean(d * d, axis=1, keepdims=True)
    out_ref[...] = d * lax.rsqrt(var + 1e-8) * lg_ref[...] + lb_ref[...]


def _tc_post(partials, loopp, h, ffn_W, h_bias, ffn_b, ln_gamma, ln_beta):
    vec = pl.BlockSpec((1, _D), lambda i: (0, 0))
    return pl.pallas_call(
        _tc_post_body,
        grid=(_G1,),
        in_specs=[
            pl.BlockSpec((_NC, _BN, _D), lambda i: (0, i, 0)),
            pl.BlockSpec((_BN, _D), lambda i: (i, 0)),
            pl.BlockSpec((_BN, _D), lambda i: (i, 0)),
            pl.BlockSpec((_D, _D), lambda i: (0, 0)),
            vec, vec, vec, vec,
        ],
        out_specs=pl.BlockSpec((_BN, _D), lambda i: (i, 0)),
        out_shape=jax.ShapeDtypeStruct((_N, _D), jnp.float32),
    )(partials, loopp, h, ffn_W, h_bias, ffn_b, ln_gamma, ln_beta)


def kernel(initial_embeddings, edge_index, etype, norm, basis, comp,
           loop_weight, h_bias, ffn_W, ffn_b, ln_gamma, ln_beta):
    h = initial_embeddings
    pad = _E_PAD - _E
    src_p = jnp.pad(edge_index[0], (0, pad))
    et_p = jnp.pad(etype, (0, pad))
    dst_p = jnp.pad(edge_index[1], (0, pad))
    nrm_p = jnp.pad(norm[:, 0], (0, pad))
    edata = jnp.stack([src_p, et_p, dst_p]).reshape(3, _NT, _CH).transpose(1, 0, 2)
    nrm2d = nrm_p.reshape(_NT, _CH)

    hw, loopp = _tc_pre(comp, h, basis, loop_weight)
    partials = _sc_edge(edata, nrm2d, hw.reshape(_R * _N, _D))
    return _tc_post(partials, loopp, h, ffn_W,
                    h_bias.reshape(1, _D), ffn_b.reshape(1, _D),
                    ln_gamma.reshape(1, _D), ln_beta.reshape(1, _D))
